# Initial kernel scaffold; baseline (speedup 1.0000x reference)
import jax
import jax.numpy as jnp
from jax.experimental import pallas as pl


def kernel(z, embeddings):
    raise NotImplementedError("write your pallas kernel here")


# speedup vs baseline: 1.8616x; 1.8616x over previous
import jax
import jax.numpy as jnp
from jax import lax
from jax.experimental import pallas as pl
from jax.experimental.pallas import tpu as pltpu


def _vq_block(z_ref, et_ref, e_ref, idx_ref, emb_ref, loss_ref):
    i = pl.program_id(0)
    zb = z_ref[...]
    et = et_ref[...]
    dot = jnp.dot(zb, et, preferred_element_type=jnp.float32)
    zsq = jnp.sum(zb * zb, axis=1, keepdims=True)
    esq = jnp.sum(et * et, axis=0, keepdims=True)
    dist = zsq - 2.0 * dot + esq
    idx = jnp.argmin(dist, axis=1).astype(jnp.int32)
    idx_ref[0, 0, :] = idx
    onehot = (lax.broadcasted_iota(jnp.int32, dist.shape, 1)
              == idx[:, None]).astype(jnp.float32)
    emb = jnp.dot(onehot, e_ref[...], preferred_element_type=jnp.float32)
    emb_ref[...] = emb
    part = jnp.sum((zb - emb) ** 2).reshape(1, 1)

    @pl.when(i == 0)
    def _():
        loss_ref[...] = jnp.zeros_like(loss_ref)

    loss_ref[...] += part


def kernel(z, embeddings):
    B, T, D = z.shape
    N = B * T
    K = embeddings.shape[0]
    BLK = 512
    NB = N // BLK
    flat = z.reshape(N, D)
    et = embeddings.T

    idx3, emb, loss_sum = pl.pallas_call(
        _vq_block,
        grid=(NB,),
        in_specs=[
            pl.BlockSpec((BLK, D), lambda i: (i, 0)),
            pl.BlockSpec((D, K), lambda i: (0, 0)),
            pl.BlockSpec((K, D), lambda i: (0, 0)),
        ],
        out_specs=[
            pl.BlockSpec((1, 1, BLK), lambda i: (i, 0, 0)),
            pl.BlockSpec((BLK, D), lambda i: (i, 0)),
            pl.BlockSpec((1, 1), lambda i: (0, 0)),
        ],
        out_shape=[
            jax.ShapeDtypeStruct((NB, 1, BLK), jnp.int32),
            jax.ShapeDtypeStruct((N, D), jnp.float32),
            jax.ShapeDtypeStruct((1, 1), jnp.float32),
        ],
    )(flat, et, embeddings)

    encoding_indices = idx3.reshape(B, T)
    emb = emb.reshape(B, T, D)
    commitment_loss = 0.25 * loss_sum[0, 0] / (N * D)
    return emb, encoding_indices, commitment_loss


# speedup vs baseline: 1.9743x; 1.0605x over previous
import jax
import jax.numpy as jnp
from jax import lax
from jax.experimental import pallas as pl
from jax.experimental.pallas import tpu as pltpu


def _vq_block(z_ref, ets_ref, e_ref, idx_ref, emb_ref, loss_ref, esq_ref):
    i = pl.program_id(0)
    K = ets_ref.shape[1]

    @pl.when(i == 0)
    def _():
        et2 = ets_ref[...]
        esq_ref[...] = 0.25 * jnp.sum(et2 * et2, axis=0, keepdims=True)
        loss_ref[...] = jnp.zeros_like(loss_ref)

    zb = z_ref[...]
    dot = jnp.dot(zb, ets_ref[...], preferred_element_type=jnp.float32)
    zsq = jnp.sum(zb * zb, axis=1, keepdims=True)
    dist = (zsq + dot) + esq_ref[...]
    minv = jnp.min(dist, axis=1, keepdims=True)
    iota = lax.broadcasted_iota(jnp.int32, dist.shape, 1)
    idx = jnp.min(jnp.where(dist == minv, iota, K), axis=1)
    idx_ref[0, 0, :] = idx
    onehot = (iota == idx[:, None]).astype(jnp.float32)
    emb_ref[...] = jnp.dot(onehot, e_ref[...],
                           preferred_element_type=jnp.float32)
    loss_ref[...] += jnp.sum(minv).reshape(1, 1)


def kernel(z, embeddings):
    B, T, D = z.shape
    N = B * T
    K = embeddings.shape[0]
    BLK = 512
    NB = N // BLK
    flat = z.reshape(N, D)
    ets = -2.0 * embeddings.T

    idx3, emb, loss_sum = pl.pallas_call(
        _vq_block,
        grid=(NB,),
        in_specs=[
            pl.BlockSpec((BLK, D), lambda i: (i, 0)),
            pl.BlockSpec((D, K), lambda i: (0, 0)),
            pl.BlockSpec((K, D), lambda i: (0, 0)),
        ],
        out_specs=[
            pl.BlockSpec((1, 1, BLK), lambda i: (i, 0, 0)),
            pl.BlockSpec((BLK, D), lambda i: (i, 0)),
            pl.BlockSpec((1, 1), lambda i: (0, 0)),
        ],
        out_shape=[
            jax.ShapeDtypeStruct((NB, 1, BLK), jnp.int32),
            jax.ShapeDtypeStruct((N, D), jnp.float32),
            jax.ShapeDtypeStruct((1, 1), jnp.float32),
        ],
        scratch_shapes=[pltpu.VMEM((1, K), jnp.float32)],
    )(flat, ets, embeddings)

    encoding_indices = idx3.reshape(B, T)
    emb = emb.reshape(B, T, D)
    commitment_loss = 0.25 * loss_sum[0, 0] / (N * D)
    return emb, encoding_indices, commitment_loss
